# Initial kernel scaffold; baseline (speedup 1.0000x reference)
import jax
import jax.numpy as jnp
from jax.experimental import pallas as pl


def kernel(hidden_a, posteriors_a):
    raise NotImplementedError("write your pallas kernel here")


# speedup vs baseline: 1.5571x; 1.5571x over previous
import functools

import jax
import jax.numpy as jnp
from jax import lax
from jax.experimental import pallas as pl
from jax.experimental.pallas import tpu as pltpu
from jax.experimental.pallas import tpu_sc as plsc

B, L, D, C = 4, 8192, 1024, 128
K = 4096
ROWS, LANES = 64, 128

NC, NS = 2, 16
NW = NC * NS
RPT = (B * K) // NW
CH = 32


def _select_body(post_ref, sel_ref, pos_ref):
    p = post_ref[...]
    ent = -jnp.sum(p * jnp.log(p + 1e-8), axis=-1)
    rel = 1.0 - ent / jnp.log(float(C))
    rel3 = rel.reshape(B, ROWS, LANES)

    u = lax.bitcast_convert_type(rel3, jnp.uint32)
    key = jnp.where(u < jnp.uint32(0x80000000),
                    u + jnp.uint32(0x80000000), ~u)

    def _cnt(x):
        s = jnp.sum(x.astype(jnp.int32), axis=2, keepdims=True)
        return jnp.sum(s, axis=1, keepdims=True)

    t = jnp.zeros((B, 1, 1), jnp.uint32)
    for i in range(31, -1, -1):
        cand = t | jnp.uint32(1 << i)
        t = jnp.where(_cnt(key >= cand) >= K, cand, t)

    gt = key > t
    eq = key == t
    quota = (K - _cnt(gt)).astype(jnp.float32)

    half = B * ROWS
    s2 = jnp.concatenate([gt.reshape(half, LANES),
                          eq.reshape(half, LANES)], axis=0
                         ).astype(jnp.float32)
    incl = (lax.broadcasted_iota(jnp.int32, (LANES, LANES), 0)
            <= lax.broadcasted_iota(jnp.int32, (LANES, LANES), 1)
            ).astype(jnp.float32)
    ones = jnp.ones((LANES, LANES), jnp.float32)
    r0 = lax.broadcasted_iota(jnp.int32, (2 * half, 2 * half), 0)
    r1 = lax.broadcasted_iota(jnp.int32, (2 * half, 2 * half), 1)
    bd = ((jnp.right_shift(r0, 6) == jnp.right_shift(r1, 6))
          & (r1 < r0)).astype(jnp.float32)
    row_incl = jnp.dot(s2, incl, preferred_element_type=jnp.float32)
    tot = jnp.dot(s2, ones, preferred_element_type=jnp.float32)
    off = jnp.dot(bd, tot, preferred_element_type=jnp.float32)
    ex = row_incl + off - s2
    cumgt = ex[:half].reshape(B, ROWS, LANES)
    eqrank = ex[half:].reshape(B, ROWS, LANES)

    sel = gt | (eq & (eqrank < quota))
    pos = cumgt + jnp.minimum(eqrank, quota)

    sel_ref[...] = sel.astype(jnp.int32)
    pos_ref[...] = pos.astype(jnp.int32)


def _tc_select(posteriors_a):
    return pl.pallas_call(
        _select_body,
        out_shape=[jax.ShapeDtypeStruct((B, ROWS, LANES), jnp.int32),
                   jax.ShapeDtypeStruct((B, ROWS, LANES), jnp.int32)],
    )(posteriors_a)


NB = 3


@functools.cache
def _make_sc_fused():
    mesh = plsc.VectorSubcoreMesh(core_axis_name="c", subcore_axis_name="s")
    nch = RPT // CH

    @functools.partial(
        pl.kernel,
        mesh=mesh,
        out_type=[jax.ShapeDtypeStruct((B * K,), jnp.int32),
                  jax.ShapeDtypeStruct((B * K, D), jnp.float32)],
        scratch_types=[pltpu.VMEM((L,), jnp.int32),
                       pltpu.VMEM((L,), jnp.int32),
                       pltpu.VMEM((K,), jnp.int32),
                       pltpu.VMEM((RPT,), jnp.int32),
                       pltpu.VMEM_SHARED((2 * K,), jnp.int32),
                       [pltpu.VMEM((CH, D), jnp.float32)] * NB,
                       [pltpu.SemaphoreType.DMA] * NB,
                       [pltpu.SemaphoreType.DMA] * NB],
        compiler_params=pltpu.CompilerParams(needs_layout_passes=False),
    )
    def _fused(sel_hbm, pos_hbm, hid_hbm, ret_hbm, out_hbm,
               sel_v, pos_v, cidx_v, idx_v, sh_idx, bufs, gsems, osems):
        cid = lax.axis_index("c")
        sid = lax.axis_index("s")
        slot = sid // 8
        myb = 2 * cid + slot
        part = sid % 8

        @pl.when(part == 0)
        def _():
            b = myb
            pltpu.sync_copy(sel_hbm.at[pl.ds(b * L, L)], sel_v)
            pltpu.sync_copy(pos_hbm.at[pl.ds(b * L, L)], pos_v)
            iota = lax.iota(jnp.int32, 16)

            def _body(j, carry):
                m = sel_v[pl.ds(j * 16, 16)]
                ps = pos_v[pl.ds(j * 16, 16)]
                plsc.store_scatter(cidx_v, [ps], iota + j * 16, mask=m != 0)
                return carry

            lax.fori_loop(0, L // 16, _body, 0)
            pltpu.sync_copy(cidx_v, sh_idx.at[pl.ds(slot * K, K)])
            pltpu.sync_copy(cidx_v, ret_hbm.at[pl.ds(b * K, K)])

        plsc.subcore_barrier()

        pltpu.sync_copy(sh_idx.at[pl.ds(slot * K + part * RPT, RPT)], idx_v)
        tab = hid_hbm.at[myb]
        obase = myb * K + part * RPT

        def _g(c):
            return pltpu.async_copy(
                tab.at[idx_v.at[pl.ds(c * CH, CH)]], bufs[c % NB],
                gsems[c % NB])

        gs = [None] * nch
        os_ = [None] * nch
        for c in range(min(NB, nch)):
            gs[c] = _g(c)
        for c in range(nch):
            gs[c].wait()
            os_[c] = pltpu.async_copy(
                bufs[c % NB], out_hbm.at[pl.ds(obase + c * CH, CH)],
                osems[c % NB])
            if c + NB < nch:
                os_[c].wait()
                gs[c + NB] = _g(c + NB)
        for c in range(max(0, nch - NB), nch):
            os_[c].wait()

    return _fused


@functools.cache
def _make_compact():
    mesh = plsc.VectorSubcoreMesh(core_axis_name="c", subcore_axis_name="s")

    @functools.partial(
        pl.kernel,
        mesh=mesh,
        out_type=[jax.ShapeDtypeStruct((B * K,), jnp.int32),
                  jax.ShapeDtypeStruct((B * K,), jnp.int32)],
        scratch_types=[pltpu.VMEM((L,), jnp.int32),
                       pltpu.VMEM((L,), jnp.int32),
                       pltpu.VMEM((K,), jnp.int32),
                       pltpu.VMEM((K,), jnp.int32)],
        compiler_params=pltpu.CompilerParams(needs_layout_passes=False),
    )
    def _compact(sel_hbm, pos_hbm, ret_hbm, flat_hbm,
                 sel_v, pos_v, ret_v, flat_v):
        wid = lax.axis_index("c") * NS + lax.axis_index("s")

        @pl.when(wid < B)
        def _():
            b = wid
            pltpu.sync_copy(sel_hbm.at[pl.ds(b * L, L)], sel_v)
            pltpu.sync_copy(pos_hbm.at[pl.ds(b * L, L)], pos_v)
            iota = lax.iota(jnp.int32, 16)

            def _body(j, carry):
                m = sel_v[pl.ds(j * 16, 16)]
                ps = pos_v[pl.ds(j * 16, 16)]
                idx = iota + j * 16
                msk = m != 0
                plsc.store_scatter(ret_v, [ps], idx, mask=msk)
                plsc.store_scatter(flat_v, [ps], idx + b * L, mask=msk)
                return carry

            lax.fori_loop(0, L // 16, _body, 0)
            pltpu.sync_copy(ret_v, ret_hbm.at[pl.ds(b * K, K)])
            pltpu.sync_copy(flat_v, flat_hbm.at[pl.ds(b * K, K)])

    return _compact


@functools.cache
def _make_gather():
    mesh = plsc.VectorSubcoreMesh(core_axis_name="c", subcore_axis_name="s")

    nch = RPT // CH

    @functools.partial(
        pl.kernel,
        mesh=mesh,
        out_type=jax.ShapeDtypeStruct((B * K, D), jnp.float32),
        scratch_types=[[pltpu.VMEM((CH,), jnp.int32)] * 2,
                       [pltpu.VMEM((CH, D), jnp.float32)] * 2,
                       [pltpu.SemaphoreType.DMA] * 2,
                       [pltpu.SemaphoreType.DMA] * 2],
    )
    def _gather(flat_hbm, hid_hbm, out_hbm, idxs, bufs, isems, gsems):
        wid = lax.axis_index("c") * NS + lax.axis_index("s")
        base = wid * RPT

        def _pre(c):
            s = c % 2
            pltpu.async_copy(flat_hbm.at[pl.ds(base + c * CH, CH)],
                             idxs[s], isems[s]).wait()
            return pltpu.async_copy(hid_hbm.at[idxs[s]], bufs[s], gsems[s])

        g = _pre(0)
        for c in range(nch):
            g.wait()
            if c + 1 < nch:
                g = _pre(c + 1)
            pltpu.sync_copy(bufs[c % 2],
                            out_hbm.at[pl.ds(base + c * CH, CH)])

    return _gather


def kernel(hidden_a, posteriors_a):
    sel, pos = _tc_select(posteriors_a)
    ret_flat, gather_idx = _make_compact()(sel.reshape(-1), pos.reshape(-1))
    out_flat = _make_gather()(gather_idx, hidden_a.reshape(B * L, D))
    hidden_a_filtered = out_flat.reshape(B, K, D)
    retained_indices = ret_flat.reshape(B, K)
    pruned_mask = jnp.ones((B, K), dtype=bool)
    return (hidden_a_filtered, retained_indices, pruned_mask)


# speedup vs baseline: 1.5729x; 1.0101x over previous
import functools

import jax
import jax.numpy as jnp
from jax import lax
from jax.experimental import pallas as pl
from jax.experimental.pallas import tpu as pltpu
from jax.experimental.pallas import tpu_sc as plsc

B, L, D, C = 4, 8192, 1024, 128
K = 4096
ROWS, LANES = 64, 128

NC, NS = 2, 16
NW = NC * NS
RPT = (B * K) // NW
CH = 32


TROWS = 16


def _rel_body(post_ref, rel_ref):
    p = post_ref[...]
    ent = -jnp.sum(p * jnp.log(p + 1e-8), axis=-1)
    rel = 1.0 - ent / jnp.log(float(C))
    rel_ref[...] = rel.reshape(1, TROWS, LANES)


def _select_body(rel_ref, sel_ref, pos_ref):
    rel3 = rel_ref[...]

    u = lax.bitcast_convert_type(rel3, jnp.uint32)
    key = jnp.where(u < jnp.uint32(0x80000000),
                    u + jnp.uint32(0x80000000), ~u)

    def _cnt(x):
        s = jnp.sum(x.astype(jnp.int32), axis=2, keepdims=True)
        return jnp.sum(s, axis=1, keepdims=True)

    t = jnp.zeros((B, 1, 1), jnp.uint32)
    for i in range(31, -1, -1):
        cand = t | jnp.uint32(1 << i)
        t = jnp.where(_cnt(key >= cand) >= K, cand, t)

    gt = key > t
    eq = key == t
    quota = (K - _cnt(gt)).astype(jnp.float32)

    half = B * ROWS
    s2 = jnp.concatenate([gt.reshape(half, LANES),
                          eq.reshape(half, LANES)], axis=0
                         ).astype(jnp.float32)
    incl = (lax.broadcasted_iota(jnp.int32, (LANES, LANES), 0)
            <= lax.broadcasted_iota(jnp.int32, (LANES, LANES), 1)
            ).astype(jnp.float32)
    ones = jnp.ones((LANES, LANES), jnp.float32)
    r0 = lax.broadcasted_iota(jnp.int32, (2 * half, 2 * half), 0)
    r1 = lax.broadcasted_iota(jnp.int32, (2 * half, 2 * half), 1)
    bd = ((jnp.right_shift(r0, 6) == jnp.right_shift(r1, 6))
          & (r1 < r0)).astype(jnp.float32)
    row_incl = jnp.dot(s2, incl, preferred_element_type=jnp.float32)
    tot = jnp.dot(s2, ones, preferred_element_type=jnp.float32)
    off = jnp.dot(bd, tot, preferred_element_type=jnp.float32)
    ex = row_incl + off - s2
    cumgt = ex[:half].reshape(B, ROWS, LANES)
    eqrank = ex[half:].reshape(B, ROWS, LANES)

    sel = gt | (eq & (eqrank < quota))
    pos = cumgt + jnp.minimum(eqrank, quota)

    sel_ref[...] = sel.astype(jnp.int32)
    pos_ref[...] = pos.astype(jnp.int32)


def _tc_select(posteriors_a):
    rel = pl.pallas_call(
        _rel_body,
        grid=(B, ROWS // TROWS),
        in_specs=[pl.BlockSpec((1, TROWS * LANES, C), lambda b, t: (b, t, 0))],
        out_specs=pl.BlockSpec((1, TROWS, LANES), lambda b, t: (b, t, 0)),
        out_shape=jax.ShapeDtypeStruct((B, ROWS, LANES), jnp.float32),
    )(posteriors_a)
    return pl.pallas_call(
        _select_body,
        out_shape=[jax.ShapeDtypeStruct((B, ROWS, LANES), jnp.int32),
                   jax.ShapeDtypeStruct((B, ROWS, LANES), jnp.int32)],
    )(rel)


NB = 3


@functools.cache
def _make_sc_fused():
    mesh = plsc.VectorSubcoreMesh(core_axis_name="c", subcore_axis_name="s")
    nch = RPT // CH

    @functools.partial(
        pl.kernel,
        mesh=mesh,
        out_type=[jax.ShapeDtypeStruct((B * K,), jnp.int32),
                  jax.ShapeDtypeStruct((B * K, D), jnp.float32)],
        scratch_types=[pltpu.VMEM((L,), jnp.int32),
                       pltpu.VMEM((L,), jnp.int32),
                       pltpu.VMEM((K,), jnp.int32),
                       pltpu.VMEM((RPT,), jnp.int32),
                       pltpu.VMEM_SHARED((2 * K,), jnp.int32),
                       [pltpu.VMEM((CH, D), jnp.float32)] * NB,
                       [pltpu.SemaphoreType.DMA] * NB,
                       [pltpu.SemaphoreType.DMA] * NB],
        compiler_params=pltpu.CompilerParams(needs_layout_passes=False),
    )
    def _fused(sel_hbm, pos_hbm, hid_hbm, ret_hbm, out_hbm,
               sel_v, pos_v, cidx_v, idx_v, sh_idx, bufs, gsems, osems):
        cid = lax.axis_index("c")
        sid = lax.axis_index("s")
        slot = sid // 8
        myb = 2 * cid + slot
        part = sid % 8

        @pl.when(part == 0)
        def _():
            b = myb
            pltpu.sync_copy(sel_hbm.at[pl.ds(b * L, L)], sel_v)
            pltpu.sync_copy(pos_hbm.at[pl.ds(b * L, L)], pos_v)
            iota = lax.iota(jnp.int32, 16)

            def _body(j, carry):
                m = sel_v[pl.ds(j * 16, 16)]
                ps = pos_v[pl.ds(j * 16, 16)]
                plsc.store_scatter(cidx_v, [ps], iota + j * 16, mask=m != 0)
                return carry

            lax.fori_loop(0, L // 16, _body, 0)
            pltpu.sync_copy(cidx_v, sh_idx.at[pl.ds(slot * K, K)])
            pltpu.sync_copy(cidx_v, ret_hbm.at[pl.ds(b * K, K)])

        plsc.subcore_barrier()

        pltpu.sync_copy(sh_idx.at[pl.ds(slot * K + part * RPT, RPT)], idx_v)
        tab = hid_hbm.at[myb]
        obase = myb * K + part * RPT

        def _g(c):
            return pltpu.async_copy(
                tab.at[idx_v.at[pl.ds(c * CH, CH)]], bufs[c % NB],
                gsems[c % NB])

        gs = [None] * nch
        os_ = [None] * nch
        for c in range(min(NB, nch)):
            gs[c] = _g(c)
        for c in range(nch):
            gs[c].wait()
            os_[c] = pltpu.async_copy(
                bufs[c % NB], out_hbm.at[pl.ds(obase + c * CH, CH)],
                osems[c % NB])
            if c + NB < nch:
                os_[c].wait()
                gs[c + NB] = _g(c + NB)
        for c in range(max(0, nch - NB), nch):
            os_[c].wait()

    return _fused


@functools.cache
def _make_compact():
    mesh = plsc.VectorSubcoreMesh(core_axis_name="c", subcore_axis_name="s")

    @functools.partial(
        pl.kernel,
        mesh=mesh,
        out_type=[jax.ShapeDtypeStruct((B * K,), jnp.int32),
                  jax.ShapeDtypeStruct((B * K,), jnp.int32)],
        scratch_types=[pltpu.VMEM((L,), jnp.int32),
                       pltpu.VMEM((L,), jnp.int32),
                       pltpu.VMEM((K,), jnp.int32),
                       pltpu.VMEM((K,), jnp.int32)],
        compiler_params=pltpu.CompilerParams(needs_layout_passes=False),
    )
    def _compact(sel_hbm, pos_hbm, ret_hbm, flat_hbm,
                 sel_v, pos_v, ret_v, flat_v):
        wid = lax.axis_index("c") * NS + lax.axis_index("s")

        @pl.when(wid < B)
        def _():
            b = wid
            pltpu.sync_copy(sel_hbm.at[pl.ds(b * L, L)], sel_v)
            pltpu.sync_copy(pos_hbm.at[pl.ds(b * L, L)], pos_v)
            iota = lax.iota(jnp.int32, 16)

            def _body(j, carry):
                m = sel_v[pl.ds(j * 16, 16)]
                ps = pos_v[pl.ds(j * 16, 16)]
                idx = iota + j * 16
                msk = m != 0
                plsc.store_scatter(ret_v, [ps], idx, mask=msk)
                plsc.store_scatter(flat_v, [ps], idx + b * L, mask=msk)
                return carry

            lax.fori_loop(0, L // 16, _body, 0)
            pltpu.sync_copy(ret_v, ret_hbm.at[pl.ds(b * K, K)])
            pltpu.sync_copy(flat_v, flat_hbm.at[pl.ds(b * K, K)])

    return _compact


@functools.cache
def _make_gather():
    mesh = plsc.VectorSubcoreMesh(core_axis_name="c", subcore_axis_name="s")

    nch = RPT // CH

    @functools.partial(
        pl.kernel,
        mesh=mesh,
        out_type=jax.ShapeDtypeStruct((B * K, D), jnp.float32),
        scratch_types=[pltpu.VMEM((RPT,), jnp.int32),
                       [pltpu.VMEM((CH, D), jnp.float32)] * 2,
                       pltpu.SemaphoreType.DMA,
                       [pltpu.SemaphoreType.DMA] * 2],
    )
    def _gather(flat_hbm, hid_hbm, out_hbm, idx_v, bufs, isem, gsems):
        wid = lax.axis_index("c") * NS + lax.axis_index("s")
        base = wid * RPT
        pltpu.async_copy(flat_hbm.at[pl.ds(base, RPT)], idx_v, isem).wait()

        def _pre(c):
            s = c % 2
            return pltpu.async_copy(
                hid_hbm.at[idx_v.at[pl.ds(c * CH, CH)]], bufs[s], gsems[s])

        g = _pre(0)
        for c in range(nch):
            g.wait()
            if c + 1 < nch:
                g = _pre(c + 1)
            pltpu.sync_copy(bufs[c % 2],
                            out_hbm.at[pl.ds(base + c * CH, CH)])

    return _gather


def kernel(hidden_a, posteriors_a):
    sel, pos = _tc_select(posteriors_a)
    ret_flat, gather_idx = _make_compact()(sel.reshape(-1), pos.reshape(-1))
    out_flat = _make_gather()(gather_idx, hidden_a.reshape(B * L, D))
    hidden_a_filtered = out_flat.reshape(B, K, D)
    retained_indices = ret_flat.reshape(B, K)
    pruned_mask = jnp.ones((B, K), dtype=bool)
    return (hidden_a_filtered, retained_indices, pruned_mask)


# speedup vs baseline: 1.5823x; 1.0060x over previous
import functools

import jax
import jax.numpy as jnp
from jax import lax
from jax.experimental import pallas as pl
from jax.experimental.pallas import tpu as pltpu
from jax.experimental.pallas import tpu_sc as plsc

B, L, D, C = 4, 8192, 1024, 128
K = 4096
ROWS, LANES = 64, 128

NC, NS = 2, 16
NW = NC * NS
RPT = (B * K) // NW
CH = 32


TROWS = 16


def _rel_body(post_ref, rel_ref):
    p = post_ref[...]
    ent = -jnp.sum(p * jnp.log(p + 1e-8), axis=-1)
    rel = 1.0 - ent / jnp.log(float(C))
    rel_ref[...] = rel.reshape(1, TROWS, LANES)


def _select_body(rel_ref, sel_ref, pos_ref):
    rel3 = rel_ref[...]

    u = lax.bitcast_convert_type(rel3, jnp.uint32)
    key = jnp.where(u < jnp.uint32(0x80000000),
                    u + jnp.uint32(0x80000000), ~u)

    def _cnt(x):
        s = jnp.sum(x.astype(jnp.int32), axis=2, keepdims=True)
        return jnp.sum(s, axis=1, keepdims=True)

    t = jnp.zeros((B, 1, 1), jnp.uint32)
    for i in range(31, -1, -1):
        cand = t | jnp.uint32(1 << i)
        t = jnp.where(_cnt(key >= cand) >= K, cand, t)

    gt = key > t
    eq = key == t
    quota = (K - _cnt(gt)).astype(jnp.float32)

    half = B * ROWS
    s2 = jnp.concatenate([gt.reshape(half, LANES),
                          eq.reshape(half, LANES)], axis=0
                         ).astype(jnp.float32)
    incl = (lax.broadcasted_iota(jnp.int32, (LANES, LANES), 0)
            <= lax.broadcasted_iota(jnp.int32, (LANES, LANES), 1)
            ).astype(jnp.float32)
    ones = jnp.ones((LANES, LANES), jnp.float32)
    r0 = lax.broadcasted_iota(jnp.int32, (2 * half, 2 * half), 0)
    r1 = lax.broadcasted_iota(jnp.int32, (2 * half, 2 * half), 1)
    bd = ((jnp.right_shift(r0, 6) == jnp.right_shift(r1, 6))
          & (r1 < r0)).astype(jnp.float32)
    row_incl = jnp.dot(s2, incl, preferred_element_type=jnp.float32)
    tot = jnp.dot(s2, ones, preferred_element_type=jnp.float32)
    off = jnp.dot(bd, tot, preferred_element_type=jnp.float32)
    ex = row_incl + off - s2
    cumgt = ex[:half].reshape(B, ROWS, LANES)
    eqrank = ex[half:].reshape(B, ROWS, LANES)

    sel = gt | (eq & (eqrank < quota))
    pos = cumgt + jnp.minimum(eqrank, quota)

    sel_ref[...] = sel.astype(jnp.int32)
    pos_ref[...] = pos.astype(jnp.int32)


def _tc_select(posteriors_a):
    rel = pl.pallas_call(
        _rel_body,
        grid=(B, ROWS // TROWS),
        in_specs=[pl.BlockSpec((1, TROWS * LANES, C), lambda b, t: (b, t, 0))],
        out_specs=pl.BlockSpec((1, TROWS, LANES), lambda b, t: (b, t, 0)),
        out_shape=jax.ShapeDtypeStruct((B, ROWS, LANES), jnp.float32),
    )(posteriors_a)
    return pl.pallas_call(
        _select_body,
        out_shape=[jax.ShapeDtypeStruct((B, ROWS, LANES), jnp.int32),
                   jax.ShapeDtypeStruct((B, ROWS, LANES), jnp.int32)],
    )(rel)


@functools.cache
def _make_sc_fused():
    mesh = plsc.VectorSubcoreMesh(core_axis_name="c", subcore_axis_name="s")
    nch = RPT // CH

    @functools.partial(
        pl.kernel,
        mesh=mesh,
        out_type=[jax.ShapeDtypeStruct((B * K,), jnp.int32),
                  jax.ShapeDtypeStruct((B * K,), jnp.int32),
                  jax.ShapeDtypeStruct((B * K, D), jnp.float32)],
        scratch_types=[pltpu.VMEM((L,), jnp.int32),
                       pltpu.VMEM((L,), jnp.int32),
                       pltpu.VMEM((K,), jnp.int32),
                       pltpu.VMEM((K,), jnp.int32),
                       pltpu.VMEM((RPT,), jnp.int32),
                       [pltpu.VMEM((CH, D), jnp.float32)] * 2,
                       pltpu.SemaphoreType.DMA,
                       [pltpu.SemaphoreType.DMA] * 2],
        compiler_params=pltpu.CompilerParams(needs_layout_passes=False),
    )
    def _fused(sel_hbm, pos_hbm, hid_hbm, ret_hbm, flat_hbm, out_hbm,
               sel_v, pos_v, ret_v, flat_v, idx_v, bufs, isem, gsems):
        cid = lax.axis_index("c")
        sid = lax.axis_index("s")
        slot = sid // 8
        myb = 2 * cid + slot
        part = sid % 8

        @pl.when(part == 0)
        def _():
            b = myb
            pltpu.sync_copy(sel_hbm.at[pl.ds(b * L, L)], sel_v)
            pltpu.sync_copy(pos_hbm.at[pl.ds(b * L, L)], pos_v)
            iota = lax.iota(jnp.int32, 16)

            def _body(j, carry):
                m = sel_v[pl.ds(j * 16, 16)]
                ps = pos_v[pl.ds(j * 16, 16)]
                idx = iota + j * 16
                msk = m != 0
                plsc.store_scatter(ret_v, [ps], idx, mask=msk)
                plsc.store_scatter(flat_v, [ps], idx + b * L, mask=msk)
                return carry

            lax.fori_loop(0, L // 16, _body, 0)
            pltpu.sync_copy(ret_v, ret_hbm.at[pl.ds(b * K, K)])
            pltpu.sync_copy(flat_v, flat_hbm.at[pl.ds(b * K, K)])

        plsc.subcore_barrier()

        base = myb * K + part * RPT
        pltpu.async_copy(flat_hbm.at[pl.ds(base, RPT)], idx_v, isem).wait()

        def _pre(c):
            s = c % 2
            return pltpu.async_copy(
                hid_hbm.at[idx_v.at[pl.ds(c * CH, CH)]], bufs[s], gsems[s])

        g = _pre(0)
        for c in range(nch):
            g.wait()
            if c + 1 < nch:
                g = _pre(c + 1)
            pltpu.sync_copy(bufs[c % 2],
                            out_hbm.at[pl.ds(base + c * CH, CH)])

    return _fused


@functools.cache
def _make_compact():
    mesh = plsc.VectorSubcoreMesh(core_axis_name="c", subcore_axis_name="s")

    @functools.partial(
        pl.kernel,
        mesh=mesh,
        out_type=[jax.ShapeDtypeStruct((B * K,), jnp.int32),
                  jax.ShapeDtypeStruct((B * K,), jnp.int32)],
        scratch_types=[pltpu.VMEM((L,), jnp.int32),
                       pltpu.VMEM((L,), jnp.int32),
                       pltpu.VMEM((K,), jnp.int32),
                       pltpu.VMEM((K,), jnp.int32)],
        compiler_params=pltpu.CompilerParams(needs_layout_passes=False),
    )
    def _compact(sel_hbm, pos_hbm, ret_hbm, flat_hbm,
                 sel_v, pos_v, ret_v, flat_v):
        wid = lax.axis_index("c") * NS + lax.axis_index("s")

        @pl.when(wid < B)
        def _():
            b = wid
            pltpu.sync_copy(sel_hbm.at[pl.ds(b * L, L)], sel_v)
            pltpu.sync_copy(pos_hbm.at[pl.ds(b * L, L)], pos_v)
            iota = lax.iota(jnp.int32, 16)

            def _body(j, carry):
                m = sel_v[pl.ds(j * 16, 16)]
                ps = pos_v[pl.ds(j * 16, 16)]
                idx = iota + j * 16
                msk = m != 0
                plsc.store_scatter(ret_v, [ps], idx, mask=msk)
                plsc.store_scatter(flat_v, [ps], idx + b * L, mask=msk)
                return carry

            lax.fori_loop(0, L // 16, _body, 0)
            pltpu.sync_copy(ret_v, ret_hbm.at[pl.ds(b * K, K)])
            pltpu.sync_copy(flat_v, flat_hbm.at[pl.ds(b * K, K)])

    return _compact


@functools.cache
def _make_gather():
    mesh = plsc.VectorSubcoreMesh(core_axis_name="c", subcore_axis_name="s")

    nch = RPT // CH

    @functools.partial(
        pl.kernel,
        mesh=mesh,
        out_type=jax.ShapeDtypeStruct((B * K, D), jnp.float32),
        scratch_types=[pltpu.VMEM((RPT,), jnp.int32),
                       [pltpu.VMEM((CH, D), jnp.float32)] * 2,
                       pltpu.SemaphoreType.DMA,
                       [pltpu.SemaphoreType.DMA] * 2],
    )
    def _gather(flat_hbm, hid_hbm, out_hbm, idx_v, bufs, isem, gsems):
        wid = lax.axis_index("c") * NS + lax.axis_index("s")
        base = wid * RPT
        pltpu.async_copy(flat_hbm.at[pl.ds(base, RPT)], idx_v, isem).wait()

        def _pre(c):
            s = c % 2
            return pltpu.async_copy(
                hid_hbm.at[idx_v.at[pl.ds(c * CH, CH)]], bufs[s], gsems[s])

        g = _pre(0)
        for c in range(nch):
            g.wait()
            if c + 1 < nch:
                g = _pre(c + 1)
            pltpu.sync_copy(bufs[c % 2],
                            out_hbm.at[pl.ds(base + c * CH, CH)])

    return _gather


def kernel(hidden_a, posteriors_a):
    sel, pos = _tc_select(posteriors_a)
    ret_flat, _, out_flat = _make_sc_fused()(
        sel.reshape(-1), pos.reshape(-1), hidden_a.reshape(B * L, D))
    hidden_a_filtered = out_flat.reshape(B, K, D)
    retained_indices = ret_flat.reshape(B, K)
    pruned_mask = jnp.ones((B, K), dtype=bool)
    return (hidden_a_filtered, retained_indices, pruned_mask)
